# baseline (speedup 1.0000x reference)
import functools

import jax
import jax.numpy as jnp
from jax import lax
from jax.experimental import pallas as pl
from jax.experimental.pallas import tpu as pltpu
from jax.experimental.pallas import tpu_sc as plsc

D = 64
NC = 2
NS = 16
NW = NC * NS
B_TOTAL = 16384 * 20
BPW = B_TOTAL // NW
CHUNK = 512
NCHUNKS = BPW // CHUNK
SUB = 128
NSUB = CHUNK // SUB
NBUF = 3
SCALE = 8.0

_mesh = plsc.VectorSubcoreMesh(
    core_axis_name="c", subcore_axis_name="s", num_cores=NC, num_subcores=NS
)


@functools.partial(
    pl.kernel,
    out_type=jax.ShapeDtypeStruct((B_TOTAL, D), jnp.float32),
    mesh=_mesh,
    compiler_params=pltpu.CompilerParams(use_tc_tiling_on_sc=False),
    scratch_types=(
        [pltpu.VMEM((NSUB, SUB), jnp.int32) for _ in range(NBUF)]
        + [pltpu.VMEM((CHUNK, D), jnp.float32) for _ in range(NBUF)]
        + [pltpu.SemaphoreType.DMA for _ in range(2 * NBUF)]
    ),
)
def _emb_lookup(tab_hbm, src_hbm, out_hbm,
                idx0, idx1, idx2, rows0, rows1, rows2,
                sg0, sg1, sg2, ss0, ss1, ss2):
    idx = (idx0, idx1, idx2)
    rows = (rows0, rows1, rows2)
    sg = (sg0, sg1, sg2)
    ss = (ss0, ss1, ss2)

    wid = lax.axis_index("s") * NC + lax.axis_index("c")
    base = wid * BPW

    def fire_gather(g, b):
        off = base + g * CHUNK
        descs = []
        for j in range(NSUB):
            pltpu.sync_copy(src_hbm.at[pl.ds(off + j * SUB, SUB)], idx[b].at[j])
            descs.append(
                pltpu.async_copy(tab_hbm.at[idx[b].at[j]],
                                 rows[b].at[pl.ds(j * SUB, SUB)], sg[b])
            )
        return descs

    def scale(b):
        r = rows[b]

        @plsc.parallel_loop(0, CHUNK, 1, unroll=8)
        def _(i):
            for c in range(D // 16):
                r[i, pl.ds(c * 16, 16)] = r[i, pl.ds(c * 16, 16)] * SCALE

    gd = [None] * NBUF
    sd = [None] * NBUF
    gd[0] = fire_gather(0, 0)
    gd[1] = fire_gather(1, 1)
    for g in range(NCHUNKS):
        b = g % NBUF
        for d in gd[b]:
            d.wait()
        scale(b)
        sd[b] = pltpu.async_copy(
            rows[b], out_hbm.at[pl.ds(base + g * CHUNK, CHUNK)], ss[b]
        )
        nxt = g + 2
        if nxt < NCHUNKS:
            nb = nxt % NBUF
            if sd[nb] is not None:
                sd[nb].wait()
            gd[nb] = fire_gather(nxt, nb)
    for b in range(NBUF):
        if sd[b] is not None:
            sd[b].wait()


def kernel(src, embedding):
    b, l = src.shape
    flat = src.reshape(b * l).astype(jnp.int32)
    out = _emb_lookup(embedding, flat)
    return out.reshape(b, l, D)


# speedup vs baseline: 1.0185x; 1.0185x over previous
import functools

import jax
import jax.numpy as jnp
from jax import lax
from jax.experimental import pallas as pl
from jax.experimental.pallas import tpu as pltpu
from jax.experimental.pallas import tpu_sc as plsc

D = 64
NC = 2
NS = 16
NW = NC * NS
B_TOTAL = 16384 * 20
BPW = B_TOTAL // NW
CHUNK = 512
NCHUNKS = BPW // CHUNK
SUB = 128
NSUB = CHUNK // SUB
NBUF = 3
SCALE = 8.0

_mesh = plsc.VectorSubcoreMesh(
    core_axis_name="c", subcore_axis_name="s", num_cores=NC, num_subcores=NS
)


ROWS_PER_W = BPW // SUB


@functools.partial(
    pl.kernel,
    out_type=jax.ShapeDtypeStruct((B_TOTAL, D), jnp.float32),
    mesh=_mesh,
    compiler_params=pltpu.CompilerParams(use_tc_tiling_on_sc=False),
    scratch_types=(
        [pltpu.VMEM((ROWS_PER_W, SUB), jnp.int32)]
        + [pltpu.VMEM((CHUNK, D), jnp.float32) for _ in range(NBUF)]
        + [pltpu.SemaphoreType.DMA for _ in range(1 + 2 * NBUF)]
    ),
)
def _emb_lookup(tab_hbm, src_hbm, out_hbm,
                idx_all, rows0, rows1, rows2,
                si, sg0, sg1, sg2, ss0, ss1, ss2):
    rows = (rows0, rows1, rows2)
    sg = (sg0, sg1, sg2)
    ss = (ss0, ss1, ss2)

    wid = lax.axis_index("s") * NC + lax.axis_index("c")
    base = wid * BPW

    pltpu.async_copy(src_hbm.at[pl.ds(wid * ROWS_PER_W, ROWS_PER_W)],
                     idx_all, si).wait()

    def fire_gather(g, b):
        descs = []
        for j in range(NSUB):
            row = g * NSUB + j
            descs.append(
                pltpu.async_copy(tab_hbm.at[idx_all.at[row]],
                                 rows[b].at[pl.ds(j * SUB, SUB)], sg[b])
            )
        return descs

    def scale(b):
        r = rows[b]

        @plsc.parallel_loop(0, CHUNK, 1, unroll=8)
        def _(i):
            for c in range(D // 16):
                r[i, pl.ds(c * 16, 16)] = r[i, pl.ds(c * 16, 16)] * SCALE

    gd = [None] * NBUF
    sd = [None] * NBUF
    gd[0] = fire_gather(0, 0)
    gd[1] = fire_gather(1, 1)
    for g in range(NCHUNKS):
        b = g % NBUF
        for d in gd[b]:
            d.wait()
        scale(b)
        sd[b] = pltpu.async_copy(
            rows[b], out_hbm.at[pl.ds(base + g * CHUNK, CHUNK)], ss[b]
        )
        nxt = g + 2
        if nxt < NCHUNKS:
            nb = nxt % NBUF
            if sd[nb] is not None:
                sd[nb].wait()
            gd[nb] = fire_gather(nxt, nb)
    for b in range(NBUF):
        if sd[b] is not None:
            sd[b].wait()


def kernel(src, embedding):
    b, l = src.shape
    src2d = src.reshape(b * l // SUB, SUB).astype(jnp.int32)
    out = _emb_lookup(embedding, src2d)
    return out.reshape(b, l, D)
